# baseline (speedup 1.0000x reference)
import jax
import jax.numpy as jnp
from jax.experimental import pallas as pl

N_OBJ = 512
N_PAIR = 2048
DIM = 128
RES = 7
HW = RES * RES
HWP = 56
U_CH = 256
OBJ_CLS = 151
REL_CLS = 51

PAIR_BLK = 64


def _obj_emb_kernel(roi_ref, logit_ref, bbox_ref, w1r_ref, w1l_ref, w1b_ref,
                    b1_ref, w2_ref, b2_ref, out_ref):
    h = (jnp.dot(roi_ref[...], w1r_ref[...], preferred_element_type=jnp.float32)
         + jnp.dot(logit_ref[...], w1l_ref[...], preferred_element_type=jnp.float32)
         + jnp.dot(bbox_ref[...], w1b_ref[...], preferred_element_type=jnp.float32)
         + b1_ref[...])
    h = jax.nn.relu(h)
    out_ref[...] = jnp.dot(h, w2_ref[...], preferred_element_type=jnp.float32) + b2_ref[...]


def _rel_kernel(u_ref, pair_ref, bbox_ref, wc_ref,
                ws_ref, wo_ref, wb_ref, bs_ref, bo_ref, bb_ref,
                w1s_ref, w1o_ref, w1b_ref, b1_ref, w2_ref, b2_ref,
                out_ref):
    M = PAIR_BLK
    HWK = HW
    sidx = pair_ref[:, 0:1]
    oidx = pair_ref[:, 1:2]
    node_iota = jax.lax.broadcasted_iota(jnp.int32, (M, N_OBJ), 1)
    oh_s = (node_iota == sidx).astype(jnp.float32)
    oh_o = (node_iota == oidx).astype(jnp.float32)
    sb = jnp.dot(oh_s, bbox_ref[...], preferred_element_type=jnp.float32)
    ob = jnp.dot(oh_o, bbox_ref[...], preferred_element_type=jnp.float32)

    ux0 = jnp.minimum(sb[:, 0:1], ob[:, 0:1])
    uy0 = jnp.minimum(sb[:, 1:2], ob[:, 1:2])
    x0s = sb[:, 0:1] - ux0
    x1s = sb[:, 2:3] - ux0
    x0o = ob[:, 0:1] - ux0
    x1o = ob[:, 2:3] - ux0
    y0s = sb[:, 1:2] - uy0
    y1s = sb[:, 3:4] - uy0
    y0o = ob[:, 1:2] - uy0
    y1o = ob[:, 3:4] - uy0
    xr = RES / jnp.maximum(x1s, x1o)
    yr = RES / jnp.maximum(y1s, y1o)

    def r2i(v, r):
        return jnp.round(v * r).astype(jnp.int32)

    xp0s, xp1s = r2i(x0s, xr), r2i(x1s, xr)
    xp0o, xp1o = r2i(x0o, xr), r2i(x1o, xr)
    yp0s, yp1s = r2i(y0s, yr), r2i(y1s, yr)
    yp0o, yp1o = r2i(y0o, yr), r2i(y1o, yr)

    hwi = jax.lax.broadcasted_iota(jnp.int32, (M, HWK), 1)
    rr = hwi // RES
    cc = hwi % RES
    smask = ((rr >= xp0s) & (rr < xp1s) & (cc >= yp0s) & (cc < yp1s)).astype(jnp.float32)
    omask = ((rr >= xp0o) & (rr < xp1o) & (cc >= yp0o) & (cc < yp1o)).astype(jnp.float32)
    bmask = jnp.maximum(1.0 - smask - omask, 0.0)

    ut = jnp.swapaxes(u_ref[...], 1, 2)
    uf = jnp.dot(ut.reshape(M * HWK, U_CH), wc_ref[...],
                 preferred_element_type=jnp.float32)
    uf3 = uf.reshape(M, HWK, DIM)

    def emb(mask, w_ref, bias_ref):
        y = (uf3 * mask[:, :, None]).reshape(M, HWK * DIM)
        out = jnp.dot(y, w_ref[...], preferred_element_type=jnp.float32)
        out = out + jnp.dot(mask, bias_ref[...], preferred_element_type=jnp.float32)
        return out

    sbj = emb(smask, ws_ref, bs_ref)
    obj = emb(omask, wo_ref, bo_ref)
    bgv = emb(bmask, wb_ref, bb_ref)

    h = jax.nn.relu(
        jnp.dot(sbj, w1s_ref[...], preferred_element_type=jnp.float32)
        + jnp.dot(obj, w1o_ref[...], preferred_element_type=jnp.float32)
        + jnp.dot(bgv, w1b_ref[...], preferred_element_type=jnp.float32)
        + b1_ref[...])
    out_ref[...] = jax.nn.relu(
        jnp.dot(h, w2_ref[...], preferred_element_type=jnp.float32) + b2_ref[...])


def _gcn_kernel(fn_ref, fe_ref, pair_ref, pairt_ref, wg_ref, bg_ref,
                wop_ref, bop_ref, wrp_ref, brp_ref, obj_out_ref, rel_out_ref):
    n, m = N_OBJ, N_PAIR
    s_row = pairt_ref[0:1, :]
    o_row = pairt_ref[1:2, :]
    s_col = pair_ref[:, 0:1]
    o_col = pair_ref[:, 1:2]

    iota_nm = jax.lax.broadcasted_iota(jnp.int32, (n, m), 0)
    Sf = (iota_nm == s_row).astype(jnp.float32)
    Of = (iota_nm == o_row).astype(jnp.float32)
    n2e = jnp.minimum(Sf + Of, 1.0)

    iota_mn = jax.lax.broadcasted_iota(jnp.int32, (m, n), 1)
    Sc = (iota_mn == s_col).astype(jnp.float32)
    Oc = (iota_mn == o_col).astype(jnp.float32)
    e2n = jnp.minimum(Sc + Oc, 1.0)

    n2n = jnp.minimum(
        jnp.dot(Sf, Oc, preferred_element_type=jnp.float32), 1.0)

    fn = fn_ref[...]
    fe = fe_ref[...]
    for l in range(4):
        W = wg_ref[l]
        b = bg_ref[l:l + 1, :]
        gn = jnp.dot(fn, W, preferred_element_type=jnp.float32)
        ge = jnp.dot(fe, W, preferred_element_type=jnp.float32)
        topn = (jnp.dot(n2n, gn, preferred_element_type=jnp.float32)
                + jnp.dot(n2e, ge, preferred_element_type=jnp.float32)
                + gn + b)
        bote = jnp.dot(e2n, gn, preferred_element_type=jnp.float32) + ge + b
        fn = fn + jax.nn.relu(topn)
        fe = fe + jax.nn.relu(bote)

    obj_out_ref[...] = jnp.dot(fn, wop_ref[...], preferred_element_type=jnp.float32) + bop_ref[...]
    rel_out_ref[...] = jnp.dot(fe, wrp_ref[...], preferred_element_type=jnp.float32) + brp_ref[...]


def kernel(roi_features, predict_logits, bboxes, union_features, params, rel_pair_idxs):
    n = roi_features.shape[0]
    m = rel_pair_idxs.shape[0]
    f32 = jnp.float32

    W1, b1 = params["obj_emb1"]
    W1r = W1[:4096]
    W1l = W1[4096:4096 + OBJ_CLS]
    W1b = W1[4096 + OBJ_CLS:]
    W2, b2 = params["obj_emb2"]

    Wc, bc = params["reduce_conv"]

    def prep_emb(p):
        W, b = p
        Wr = W.reshape(DIM, HW, DIM).transpose(1, 0, 2)
        Wp = Wr.reshape(HW * DIM, DIM)
        Bias = jnp.einsum("d,hdo->ho", bc, Wr)
        return Wp, Bias, b

    Ws, Bs, bs = prep_emb(params["sbj_emb"])
    Wo, Bo, bo = prep_emb(params["obj_emb"])
    Wb, Bb, bb = prep_emb(params["bg_emb"])

    Wc1, bc1 = params["comp1"]
    W1s, W1o, W1bg = Wc1[:DIM], Wc1[DIM:2 * DIM], Wc1[2 * DIM:]
    b1_eff = bs @ W1s + bo @ W1o + bb @ W1bg + bc1
    Wc2, bc2 = params["comp2"]

    Wg = jnp.stack([w for w, _ in params["gcn"]])
    Bg = jnp.stack([b for _, b in params["gcn"]])
    Wop, bop = params["obj_pred"]
    Wrp, brp = params["rel_pred"]

    U = union_features.reshape(m, U_CH, HW)
    pair = rel_pair_idxs.astype(jnp.int32)
    pair_t = pair.T

    row = lambda v: v.reshape(1, -1)

    obj_feats = pl.pallas_call(
        _obj_emb_kernel,
        out_shape=jax.ShapeDtypeStruct((n, DIM), f32),
    )(roi_features, predict_logits, bboxes, W1r, W1l, W1b, row(b1), W2, row(b2))

    G = m // PAIR_BLK
    rel_feats = pl.pallas_call(
        _rel_kernel,
        grid=(G,),
        in_specs=[
            pl.BlockSpec((PAIR_BLK, U_CH, HW), lambda i: (i, 0, 0)),
            pl.BlockSpec((PAIR_BLK, 2), lambda i: (i, 0)),
            pl.BlockSpec((n, 4), lambda i: (0, 0)),
            pl.BlockSpec((U_CH, DIM), lambda i: (0, 0)),
            pl.BlockSpec((HW * DIM, DIM), lambda i: (0, 0)),
            pl.BlockSpec((HW * DIM, DIM), lambda i: (0, 0)),
            pl.BlockSpec((HW * DIM, DIM), lambda i: (0, 0)),
            pl.BlockSpec((HW, DIM), lambda i: (0, 0)),
            pl.BlockSpec((HW, DIM), lambda i: (0, 0)),
            pl.BlockSpec((HW, DIM), lambda i: (0, 0)),
            pl.BlockSpec((DIM, DIM), lambda i: (0, 0)),
            pl.BlockSpec((DIM, DIM), lambda i: (0, 0)),
            pl.BlockSpec((DIM, DIM), lambda i: (0, 0)),
            pl.BlockSpec((1, DIM), lambda i: (0, 0)),
            pl.BlockSpec((DIM, DIM), lambda i: (0, 0)),
            pl.BlockSpec((1, DIM), lambda i: (0, 0)),
        ],
        out_specs=pl.BlockSpec((PAIR_BLK, DIM), lambda i: (i, 0)),
        out_shape=jax.ShapeDtypeStruct((m, DIM), f32),
    )(U, pair, bboxes, Wc, Ws, Wo, Wb, Bs, Bo, Bb,
      W1s, W1o, W1bg, row(b1_eff), Wc2, row(bc2))

    obj_out, rel_out = pl.pallas_call(
        _gcn_kernel,
        out_shape=(jax.ShapeDtypeStruct((n, OBJ_CLS), f32),
                   jax.ShapeDtypeStruct((m, REL_CLS), f32)),
    )(obj_feats, rel_feats, pair, pair_t, Wg, Bg,
      Wop, row(bop), Wrp, row(brp))

    return obj_out, rel_out


# speedup vs baseline: 1.1997x; 1.1997x over previous
import jax
import jax.numpy as jnp
from jax.experimental import pallas as pl

N_OBJ = 512
N_PAIR = 2048
DIM = 128
RES = 7
HW = RES * RES
HWP = 56
U_CH = 256
OBJ_CLS = 151
REL_CLS = 51

PAIR_BLK = 128


def _obj_emb_kernel(roi_ref, logit_ref, bbox_ref, w1r_ref, w1l_ref, w1b_ref,
                    b1_ref, w2_ref, b2_ref, out_ref):
    h = (jnp.dot(roi_ref[...], w1r_ref[...], preferred_element_type=jnp.float32)
         + jnp.dot(logit_ref[...], w1l_ref[...], preferred_element_type=jnp.float32)
         + jnp.dot(bbox_ref[...], w1b_ref[...], preferred_element_type=jnp.float32)
         + b1_ref[...])
    h = jax.nn.relu(h)
    out_ref[...] = jnp.dot(h, w2_ref[...], preferred_element_type=jnp.float32) + b2_ref[...]


def _rel_kernel(u_ref, pair_ref, bbox_ref, t_ref,
                ws_ref, wo_ref, wb_ref, bs_ref, bo_ref, bb_ref,
                w1s_ref, w1o_ref, w1b_ref, b1_ref, w2_ref, b2_ref,
                out_ref):
    M = PAIR_BLK
    sidx = pair_ref[:, 0:1]
    oidx = pair_ref[:, 1:2]
    node_iota = jax.lax.broadcasted_iota(jnp.int32, (M, N_OBJ), 1)
    oh_s = (node_iota == sidx).astype(jnp.float32)
    oh_o = (node_iota == oidx).astype(jnp.float32)
    sb = jnp.dot(oh_s, bbox_ref[...], preferred_element_type=jnp.float32)
    ob = jnp.dot(oh_o, bbox_ref[...], preferred_element_type=jnp.float32)

    ux0 = jnp.minimum(sb[:, 0:1], ob[:, 0:1])
    uy0 = jnp.minimum(sb[:, 1:2], ob[:, 1:2])
    x0s = sb[:, 0:1] - ux0
    x1s = sb[:, 2:3] - ux0
    x0o = ob[:, 0:1] - ux0
    x1o = ob[:, 2:3] - ux0
    y0s = sb[:, 1:2] - uy0
    y1s = sb[:, 3:4] - uy0
    y0o = ob[:, 1:2] - uy0
    y1o = ob[:, 3:4] - uy0
    xr = RES / jnp.maximum(x1s, x1o)
    yr = RES / jnp.maximum(y1s, y1o)

    def r2i(v, r):
        return jnp.round(v * r).astype(jnp.int32)

    xp0s, xp1s = r2i(x0s, xr), r2i(x1s, xr)
    xp0o, xp1o = r2i(x0o, xr), r2i(x1o, xr)
    yp0s, yp1s = r2i(y0s, yr), r2i(y1s, yr)
    yp0o, yp1o = r2i(y0o, yr), r2i(y1o, yr)

    hwi = jax.lax.broadcasted_iota(jnp.int32, (M, HWP), 1)
    rr = hwi // RES
    cc = hwi % RES
    smask = ((rr >= xp0s) & (rr < xp1s) & (cc >= yp0s) & (cc < yp1s)).astype(jnp.float32)
    omask = ((rr >= xp0o) & (rr < xp1o) & (cc >= yp0o) & (cc < yp1o)).astype(jnp.float32)
    bmask = jnp.maximum(1.0 - smask - omask, 0.0) * (hwi < HW).astype(jnp.float32)

    ub = u_ref[...].astype(jnp.bfloat16)

    def emb(mask, w_ref, bias_ref):
        mask_e = jnp.dot(mask.astype(jnp.bfloat16), t_ref[...],
                         preferred_element_type=jnp.float32).astype(jnp.bfloat16)
        y = ub * mask_e
        out = jnp.dot(y, w_ref[...], preferred_element_type=jnp.float32)
        out = out + jnp.dot(mask, bias_ref[...], preferred_element_type=jnp.float32)
        return out

    sbj = emb(smask, ws_ref, bs_ref)
    obj = emb(omask, wo_ref, bo_ref)
    bgv = emb(bmask, wb_ref, bb_ref)

    h = jax.nn.relu(
        jnp.dot(sbj, w1s_ref[...], preferred_element_type=jnp.float32)
        + jnp.dot(obj, w1o_ref[...], preferred_element_type=jnp.float32)
        + jnp.dot(bgv, w1b_ref[...], preferred_element_type=jnp.float32)
        + b1_ref[...])
    out_ref[...] = jax.nn.relu(
        jnp.dot(h, w2_ref[...], preferred_element_type=jnp.float32) + b2_ref[...])


def _gcn_kernel(fn_ref, fe_ref, pair_ref, pairt_ref, wg_ref, bg_ref,
                wop_ref, bop_ref, wrp_ref, brp_ref, obj_out_ref, rel_out_ref):
    n, m = N_OBJ, N_PAIR
    s_row = pairt_ref[0:1, :]
    o_row = pairt_ref[1:2, :]
    s_col = pair_ref[:, 0:1]
    o_col = pair_ref[:, 1:2]

    iota_nm = jax.lax.broadcasted_iota(jnp.int32, (n, m), 0)
    Sf = (iota_nm == s_row).astype(jnp.float32)
    Of = (iota_nm == o_row).astype(jnp.float32)
    n2e = jnp.minimum(Sf + Of, 1.0)

    iota_mn = jax.lax.broadcasted_iota(jnp.int32, (m, n), 1)
    Sc = (iota_mn == s_col).astype(jnp.float32)
    Oc = (iota_mn == o_col).astype(jnp.float32)
    e2n = jnp.minimum(Sc + Oc, 1.0)

    n2n = jnp.minimum(
        jnp.dot(Sf, Oc, preferred_element_type=jnp.float32), 1.0)

    fn = fn_ref[...]
    fe = fe_ref[...]
    for l in range(4):
        W = wg_ref[l]
        b = bg_ref[l:l + 1, :]
        gn = jnp.dot(fn, W, preferred_element_type=jnp.float32)
        ge = jnp.dot(fe, W, preferred_element_type=jnp.float32)
        topn = (jnp.dot(n2n, gn, preferred_element_type=jnp.float32)
                + jnp.dot(n2e, ge, preferred_element_type=jnp.float32)
                + gn + b)
        bote = jnp.dot(e2n, gn, preferred_element_type=jnp.float32) + ge + b
        fn = fn + jax.nn.relu(topn)
        fe = fe + jax.nn.relu(bote)

    obj_out_ref[...] = jnp.dot(fn, wop_ref[...], preferred_element_type=jnp.float32) + bop_ref[...]
    rel_out_ref[...] = jnp.dot(fe, wrp_ref[...], preferred_element_type=jnp.float32) + brp_ref[...]


def kernel(roi_features, predict_logits, bboxes, union_features, params, rel_pair_idxs):
    n = roi_features.shape[0]
    m = rel_pair_idxs.shape[0]
    f32 = jnp.float32

    W1, b1 = params["obj_emb1"]
    W1r = W1[:4096]
    W1l = W1[4096:4096 + OBJ_CLS]
    W1b = W1[4096 + OBJ_CLS:]
    W2, b2 = params["obj_emb2"]

    Wc, bc = params["reduce_conv"]

    def prep_emb(p):
        W, b = p
        W3 = W.reshape(DIM, HW, DIM)
        A = jnp.einsum("cd,dho->cho", Wc, W3).reshape(U_CH * HW, DIM)
        Bias = jnp.einsum("d,dho->ho", bc, W3)
        Bp = jnp.pad(Bias, ((0, HWP - HW), (0, 0)))
        return A.astype(jnp.bfloat16), Bp, b

    Ws, Bs, bs = prep_emb(params["sbj_emb"])
    Wo, Bo, bo = prep_emb(params["obj_emb"])
    Wb, Bb, bb = prep_emb(params["bg_emb"])

    hh = jnp.arange(HWP)[:, None]
    hp = jnp.tile(jnp.arange(HW)[None, :], (1, U_CH)).reshape(1, U_CH * HW)
    T_exp = (hh == hp).astype(jnp.bfloat16)

    Wc1, bc1 = params["comp1"]
    W1s, W1o, W1bg = Wc1[:DIM], Wc1[DIM:2 * DIM], Wc1[2 * DIM:]
    b1_eff = bs @ W1s + bo @ W1o + bb @ W1bg + bc1
    Wc2, bc2 = params["comp2"]

    Wg = jnp.stack([w for w, _ in params["gcn"]])
    Bg = jnp.stack([b for _, b in params["gcn"]])
    Wop, bop = params["obj_pred"]
    Wrp, brp = params["rel_pred"]

    U = union_features.reshape(m, U_CH * HW)
    pair = rel_pair_idxs.astype(jnp.int32)
    pair_t = pair.T

    row = lambda v: v.reshape(1, -1)

    obj_feats = pl.pallas_call(
        _obj_emb_kernel,
        out_shape=jax.ShapeDtypeStruct((n, DIM), f32),
    )(roi_features, predict_logits, bboxes, W1r, W1l, W1b, row(b1), W2, row(b2))

    G = m // PAIR_BLK
    rel_feats = pl.pallas_call(
        _rel_kernel,
        grid=(G,),
        in_specs=[
            pl.BlockSpec((PAIR_BLK, U_CH * HW), lambda i: (i, 0)),
            pl.BlockSpec((PAIR_BLK, 2), lambda i: (i, 0)),
            pl.BlockSpec((n, 4), lambda i: (0, 0)),
            pl.BlockSpec((HWP, U_CH * HW), lambda i: (0, 0)),
            pl.BlockSpec((U_CH * HW, DIM), lambda i: (0, 0)),
            pl.BlockSpec((U_CH * HW, DIM), lambda i: (0, 0)),
            pl.BlockSpec((U_CH * HW, DIM), lambda i: (0, 0)),
            pl.BlockSpec((HWP, DIM), lambda i: (0, 0)),
            pl.BlockSpec((HWP, DIM), lambda i: (0, 0)),
            pl.BlockSpec((HWP, DIM), lambda i: (0, 0)),
            pl.BlockSpec((DIM, DIM), lambda i: (0, 0)),
            pl.BlockSpec((DIM, DIM), lambda i: (0, 0)),
            pl.BlockSpec((DIM, DIM), lambda i: (0, 0)),
            pl.BlockSpec((1, DIM), lambda i: (0, 0)),
            pl.BlockSpec((DIM, DIM), lambda i: (0, 0)),
            pl.BlockSpec((1, DIM), lambda i: (0, 0)),
        ],
        out_specs=pl.BlockSpec((PAIR_BLK, DIM), lambda i: (i, 0)),
        out_shape=jax.ShapeDtypeStruct((m, DIM), f32),
    )(U, pair, bboxes, T_exp, Ws, Wo, Wb, Bs, Bo, Bb,
      W1s, W1o, W1bg, row(b1_eff), Wc2, row(bc2))

    obj_out, rel_out = pl.pallas_call(
        _gcn_kernel,
        out_shape=(jax.ShapeDtypeStruct((n, OBJ_CLS), f32),
                   jax.ShapeDtypeStruct((m, REL_CLS), f32)),
    )(obj_feats, rel_feats, pair, pair_t, Wg, Bg,
      Wop, row(bop), Wrp, row(brp))

    return obj_out, rel_out


# speedup vs baseline: 1.7668x; 1.4727x over previous
import jax
import jax.numpy as jnp
from jax.experimental import pallas as pl

N_OBJ = 512
N_PAIR = 2048
DIM = 128
RES = 7
HW = RES * RES
HWP = 56
U_CH = 256
OBJ_CLS = 151
REL_CLS = 51

PAIR_BLK = 128


def _obj_emb_kernel(roi_ref, logit_ref, bbox_ref, w1r_ref, w1l_ref, w1b_ref,
                    b1_ref, w2_ref, b2_ref, out_ref):
    h = (jnp.dot(roi_ref[...], w1r_ref[...], preferred_element_type=jnp.float32)
         + jnp.dot(logit_ref[...], w1l_ref[...], preferred_element_type=jnp.float32)
         + jnp.dot(bbox_ref[...], w1b_ref[...], preferred_element_type=jnp.float32)
         + b1_ref[...])
    h = jax.nn.relu(h)
    out_ref[...] = jnp.dot(h, w2_ref[...], preferred_element_type=jnp.float32) + b2_ref[...]


def _rel_kernel(u_ref, pair_ref, bbox_ref, wc_ref,
                ws_ref, wo_ref, wb_ref, bs_ref, bo_ref, bb_ref,
                w1s_ref, w1o_ref, w1b_ref, b1_ref, w2_ref, b2_ref,
                out_ref):
    M = PAIR_BLK
    sidx = pair_ref[:, 0:1]
    oidx = pair_ref[:, 1:2]
    node_iota = jax.lax.broadcasted_iota(jnp.int32, (M, N_OBJ), 1)
    oh_s = (node_iota == sidx).astype(jnp.float32)
    oh_o = (node_iota == oidx).astype(jnp.float32)
    sb = jnp.dot(oh_s, bbox_ref[...], preferred_element_type=jnp.float32)
    ob = jnp.dot(oh_o, bbox_ref[...], preferred_element_type=jnp.float32)

    ux0 = jnp.minimum(sb[:, 0:1], ob[:, 0:1])
    uy0 = jnp.minimum(sb[:, 1:2], ob[:, 1:2])
    x0s = sb[:, 0:1] - ux0
    x1s = sb[:, 2:3] - ux0
    x0o = ob[:, 0:1] - ux0
    x1o = ob[:, 2:3] - ux0
    y0s = sb[:, 1:2] - uy0
    y1s = sb[:, 3:4] - uy0
    y0o = ob[:, 1:2] - uy0
    y1o = ob[:, 3:4] - uy0
    xr = RES / jnp.maximum(x1s, x1o)
    yr = RES / jnp.maximum(y1s, y1o)

    def r2i(v, r):
        return jnp.round(v * r).astype(jnp.int32)

    xp0s, xp1s = r2i(x0s, xr), r2i(x1s, xr)
    xp0o, xp1o = r2i(x0o, xr), r2i(x1o, xr)
    yp0s, yp1s = r2i(y0s, yr), r2i(y1s, yr)
    yp0o, yp1o = r2i(y0o, yr), r2i(y1o, yr)

    hwi = jax.lax.broadcasted_iota(jnp.int32, (M, HWP), 1)
    rr = hwi // RES
    cc = hwi % RES
    smask = ((rr >= xp0s) & (rr < xp1s) & (cc >= yp0s) & (cc < yp1s)).astype(jnp.float32)
    omask = ((rr >= xp0o) & (rr < xp1o) & (cc >= yp0o) & (cc < yp1o)).astype(jnp.float32)
    bmask = jnp.maximum(1.0 - smask - omask, 0.0) * (hwi < HW).astype(jnp.float32)

    uf = jnp.dot(u_ref[...], wc_ref[...], preferred_element_type=jnp.float32)
    uf3 = uf.reshape(M, HWP, DIM)

    def emb(mask, w_ref, bias_ref):
        y = (uf3 * mask[:, :, None]).reshape(M, HWP * DIM)
        out = jnp.dot(y, w_ref[...], preferred_element_type=jnp.float32)
        out = out + jnp.dot(mask, bias_ref[...], preferred_element_type=jnp.float32)
        return out

    sbj = emb(smask, ws_ref, bs_ref)
    obj = emb(omask, wo_ref, bo_ref)
    bgv = emb(bmask, wb_ref, bb_ref)

    h = jax.nn.relu(
        jnp.dot(sbj, w1s_ref[...], preferred_element_type=jnp.float32)
        + jnp.dot(obj, w1o_ref[...], preferred_element_type=jnp.float32)
        + jnp.dot(bgv, w1b_ref[...], preferred_element_type=jnp.float32)
        + b1_ref[...])
    out_ref[...] = jax.nn.relu(
        jnp.dot(h, w2_ref[...], preferred_element_type=jnp.float32) + b2_ref[...])


def _gcn_kernel(fn_ref, fe_ref, pair_ref, pairt_ref, wg_ref, bg_ref,
                wop_ref, bop_ref, wrp_ref, brp_ref, obj_out_ref, rel_out_ref):
    n, m = N_OBJ, N_PAIR
    s_row = pairt_ref[0:1, :]
    o_row = pairt_ref[1:2, :]
    s_col = pair_ref[:, 0:1]
    o_col = pair_ref[:, 1:2]

    iota_nm = jax.lax.broadcasted_iota(jnp.int32, (n, m), 0)
    Sf = (iota_nm == s_row).astype(jnp.float32)
    Of = (iota_nm == o_row).astype(jnp.float32)
    n2e = jnp.minimum(Sf + Of, 1.0)

    iota_mn = jax.lax.broadcasted_iota(jnp.int32, (m, n), 1)
    Sc = (iota_mn == s_col).astype(jnp.float32)
    Oc = (iota_mn == o_col).astype(jnp.float32)
    e2n = jnp.minimum(Sc + Oc, 1.0)

    n2n = jnp.minimum(
        jnp.dot(Sf, Oc, preferred_element_type=jnp.float32), 1.0)

    fn = fn_ref[...]
    fe = fe_ref[...]
    for l in range(4):
        W = wg_ref[l]
        b = bg_ref[l:l + 1, :]
        gn = jnp.dot(fn, W, preferred_element_type=jnp.float32)
        ge = jnp.dot(fe, W, preferred_element_type=jnp.float32)
        topn = (jnp.dot(n2n, gn, preferred_element_type=jnp.float32)
                + jnp.dot(n2e, ge, preferred_element_type=jnp.float32)
                + gn + b)
        bote = jnp.dot(e2n, gn, preferred_element_type=jnp.float32) + ge + b
        fn = fn + jax.nn.relu(topn)
        fe = fe + jax.nn.relu(bote)

    obj_out_ref[...] = jnp.dot(fn, wop_ref[...], preferred_element_type=jnp.float32) + bop_ref[...]
    rel_out_ref[...] = jnp.dot(fe, wrp_ref[...], preferred_element_type=jnp.float32) + brp_ref[...]


def kernel(roi_features, predict_logits, bboxes, union_features, params, rel_pair_idxs):
    n = roi_features.shape[0]
    m = rel_pair_idxs.shape[0]
    f32 = jnp.float32

    W1, b1 = params["obj_emb1"]
    W1r = W1[:4096]
    W1l = W1[4096:4096 + OBJ_CLS]
    W1b = W1[4096 + OBJ_CLS:]
    W2, b2 = params["obj_emb2"]

    Wc, bc = params["reduce_conv"]

    def prep_emb(p):
        W, b = p
        Wr = W.reshape(DIM, HW, DIM).transpose(1, 0, 2)
        Wp = jnp.pad(Wr, ((0, HWP - HW), (0, 0), (0, 0))).reshape(HWP * DIM, DIM)
        Bias = jnp.einsum("d,hdo->ho", bc, Wr)
        Bp = jnp.pad(Bias, ((0, HWP - HW), (0, 0)))
        return Wp, Bp, b

    Ws, Bs, bs = prep_emb(params["sbj_emb"])
    Wo, Bo, bo = prep_emb(params["obj_emb"])
    Wb, Bb, bb = prep_emb(params["bg_emb"])

    Wc1, bc1 = params["comp1"]
    W1s, W1o, W1bg = Wc1[:DIM], Wc1[DIM:2 * DIM], Wc1[2 * DIM:]
    b1_eff = bs @ W1s + bo @ W1o + bb @ W1bg + bc1
    Wc2, bc2 = params["comp2"]

    Wg = jnp.stack([w for w, _ in params["gcn"]])
    Bg = jnp.stack([b for _, b in params["gcn"]])
    Wop, bop = params["obj_pred"]
    Wrp, brp = params["rel_pred"]

    U = union_features.reshape(m, U_CH, HW).astype(jnp.bfloat16).swapaxes(1, 2)
    U = jnp.pad(U, ((0, 0), (0, HWP - HW), (0, 0))).reshape(m * HWP, U_CH)
    pair = rel_pair_idxs.astype(jnp.int32)
    pair_t = pair.T

    row = lambda v: v.reshape(1, -1)

    obj_feats = pl.pallas_call(
        _obj_emb_kernel,
        out_shape=jax.ShapeDtypeStruct((n, DIM), f32),
    )(roi_features, predict_logits, bboxes, W1r, W1l, W1b, row(b1), W2, row(b2))

    G = m // PAIR_BLK
    rel_feats = pl.pallas_call(
        _rel_kernel,
        grid=(G,),
        in_specs=[
            pl.BlockSpec((PAIR_BLK * HWP, U_CH), lambda i: (i, 0)),
            pl.BlockSpec((PAIR_BLK, 2), lambda i: (i, 0)),
            pl.BlockSpec((n, 4), lambda i: (0, 0)),
            pl.BlockSpec((U_CH, DIM), lambda i: (0, 0)),
            pl.BlockSpec((HWP * DIM, DIM), lambda i: (0, 0)),
            pl.BlockSpec((HWP * DIM, DIM), lambda i: (0, 0)),
            pl.BlockSpec((HWP * DIM, DIM), lambda i: (0, 0)),
            pl.BlockSpec((HWP, DIM), lambda i: (0, 0)),
            pl.BlockSpec((HWP, DIM), lambda i: (0, 0)),
            pl.BlockSpec((HWP, DIM), lambda i: (0, 0)),
            pl.BlockSpec((DIM, DIM), lambda i: (0, 0)),
            pl.BlockSpec((DIM, DIM), lambda i: (0, 0)),
            pl.BlockSpec((DIM, DIM), lambda i: (0, 0)),
            pl.BlockSpec((1, DIM), lambda i: (0, 0)),
            pl.BlockSpec((DIM, DIM), lambda i: (0, 0)),
            pl.BlockSpec((1, DIM), lambda i: (0, 0)),
        ],
        out_specs=pl.BlockSpec((PAIR_BLK, DIM), lambda i: (i, 0)),
        out_shape=jax.ShapeDtypeStruct((m, DIM), f32),
    )(U, pair, bboxes, Wc.astype(jnp.bfloat16), Ws, Wo, Wb, Bs, Bo, Bb,
      W1s, W1o, W1bg, row(b1_eff), Wc2, row(bc2))

    obj_out, rel_out = pl.pallas_call(
        _gcn_kernel,
        out_shape=(jax.ShapeDtypeStruct((n, OBJ_CLS), f32),
                   jax.ShapeDtypeStruct((m, REL_CLS), f32)),
    )(obj_feats, rel_feats, pair, pair_t, Wg, Bg,
      Wop, row(bop), Wrp, row(brp))

    return obj_out, rel_out
